# Initial kernel scaffold; baseline (speedup 1.0000x reference)
import jax
import jax.numpy as jnp
from jax.experimental import pallas as pl


def kernel(x, codebook):
    raise NotImplementedError("write your pallas kernel here")


# speedup vs baseline: 2.7002x; 2.7002x over previous
import functools
import math

import jax
import jax.numpy as jnp
from jax.experimental import pallas as pl
from jax.experimental.pallas import tpu as pltpu

_BIG = 1e30
_NT = 1024


def _knn_body(x_ref, cb_ref, o_ref, best_v, best_i, *, n_total, nt, num_tiles):
    t = pl.program_id(0)
    x = x_ref[...]
    c = cb_ref[...]
    q = x.shape[0]

    xc = jax.lax.dot_general(
        x, c, (((1,), (1,)), ((), ())), preferred_element_type=jnp.float32
    )
    c_sq = jnp.sum(c * c, axis=1)
    x_sq = jnp.sum(x * x, axis=1, keepdims=True)
    d = x_sq - 2.0 * xc + c_sq[None, :]

    col = t * nt + jax.lax.broadcasted_iota(jnp.int32, (q, nt), 1)
    d = jnp.where(col < n_total, d, _BIG)

    tmin = jnp.min(d, axis=1, keepdims=True)
    targ = jnp.min(
        jnp.where(d == tmin, col, jnp.int32(2**31 - 1)), axis=1, keepdims=True
    )

    @pl.when(t == 0)
    def _():
        best_v[...] = tmin
        best_i[...] = targ

    @pl.when(t > 0)
    def _():
        upd = tmin < best_v[...]
        best_v[...] = jnp.where(upd, tmin, best_v[...])
        best_i[...] = jnp.where(upd, targ, best_i[...])

    @pl.when(t == num_tiles - 1)
    def _():
        o_ref[...] = best_i[...]


def kernel(x, codebook):
    q, d_dim = x.shape
    n = codebook.shape[0]
    nt = _NT
    num_tiles = math.ceil(n / nt)

    body = functools.partial(_knn_body, n_total=n, nt=nt, num_tiles=num_tiles)
    out = pl.pallas_call(
        body,
        grid=(num_tiles,),
        in_specs=[
            pl.BlockSpec((q, d_dim), lambda i: (0, 0)),
            pl.BlockSpec((nt, d_dim), lambda i: (i, 0)),
        ],
        out_specs=pl.BlockSpec((q, 1), lambda i: (0, 0)),
        out_shape=jax.ShapeDtypeStruct((q, 1), jnp.int32),
        scratch_shapes=[
            pltpu.VMEM((q, 1), jnp.float32),
            pltpu.VMEM((q, 1), jnp.int32),
        ],
    )(x, codebook)
    return out


# speedup vs baseline: 3.8663x; 1.4319x over previous
import functools

import jax
import jax.numpy as jnp
from jax.experimental import pallas as pl
from jax.experimental.pallas import tpu as pltpu

_BIG = 1e30
_IMAX = 2**31 - 1
_NT = 1600
_QG = 8


def _p1_body(x_ref, cb_ref, bt_ref, best_v, best_t, *, num_tiles):
    t = pl.program_id(0)
    x = x_ref[...]
    c = cb_ref[...]
    xm2 = x * -2.0
    x_sq = jnp.sum(x * x, axis=1, keepdims=True)
    m = jax.lax.dot_general(
        xm2, c, (((1,), (1,)), ((), ())), preferred_element_type=jnp.float32
    )
    c_sq = jnp.sum(c * c, axis=1)
    d = (x_sq + m) + c_sq[None, :]
    tmin = jnp.min(d, axis=1, keepdims=True)

    @pl.when(t == 0)
    def _():
        best_v[...] = tmin
        best_t[...] = jnp.zeros_like(best_t)

    @pl.when(t > 0)
    def _():
        upd = tmin < best_v[...]
        best_v[...] = jnp.where(upd, tmin, best_v[...])
        best_t[...] = jnp.where(upd, t, best_t[...])

    @pl.when(t == num_tiles - 1)
    def _():
        bt_ref[...] = best_t[...]


def _p2_dma(cb_hbm, buf, sems, bt_smem, g, i, nt, qg):
    start = bt_smem[g * qg + i, 0] * nt
    return pltpu.make_async_copy(
        cb_hbm.at[pl.ds(start, nt), :],
        buf.at[g % 2, pl.ds(i * nt, nt), :],
        sems.at[g % 2, i],
    )


def _p2_body(bt_smem, x_ref, bt_ref, cb_hbm, o_ref, buf, sems, *, nt, qg,
             num_groups):
    g = pl.program_id(0)

    @pl.when(g == 0)
    def _():
        for i in range(qg):
            _p2_dma(cb_hbm, buf, sems, bt_smem, 0, i, nt, qg).start()

    @pl.when(g + 1 < num_groups)
    def _():
        for i in range(qg):
            _p2_dma(cb_hbm, buf, sems, bt_smem, g + 1, i, nt, qg).start()

    for i in range(qg):
        _p2_dma(cb_hbm, buf, sems, bt_smem, g, i, nt, qg).wait()

    x = x_ref[...]
    cg = buf[g % 2]
    xm2 = x * -2.0
    x_sq = jnp.sum(x * x, axis=1, keepdims=True)
    m = jax.lax.dot_general(
        xm2, cg, (((1,), (1,)), ((), ())), preferred_element_type=jnp.float32
    )
    c_sq = jnp.sum(cg * cg, axis=1)
    d = (x_sq + m) + c_sq[None, :]

    row = jax.lax.broadcasted_iota(jnp.int32, (qg, qg * nt), 0)
    col = jax.lax.broadcasted_iota(jnp.int32, (qg, qg * nt), 1)
    lo = row * nt
    valid = (col >= lo) & (col < lo + nt)
    dm = jnp.where(valid, d, _BIG)
    tmin = jnp.min(dm, axis=1, keepdims=True)

    gcol = (col - lo) + bt_ref[...] * nt
    idx = jnp.min(
        jnp.where(dm == tmin, gcol, _IMAX), axis=1, keepdims=True
    )
    o_ref[...] = idx


def kernel(x, codebook):
    q, d_dim = x.shape
    n = codebook.shape[0]
    nt = _NT
    qg = _QG
    num_tiles = n // nt
    assert num_tiles * nt == n
    num_groups = q // qg

    bt = pl.pallas_call(
        functools.partial(_p1_body, num_tiles=num_tiles),
        grid=(num_tiles,),
        in_specs=[
            pl.BlockSpec((q, d_dim), lambda i: (0, 0)),
            pl.BlockSpec((nt, d_dim), lambda i: (i, 0)),
        ],
        out_specs=pl.BlockSpec((q, 1), lambda i: (0, 0)),
        out_shape=jax.ShapeDtypeStruct((q, 1), jnp.int32),
        scratch_shapes=[
            pltpu.VMEM((q, 1), jnp.float32),
            pltpu.VMEM((q, 1), jnp.int32),
        ],
    )(x, codebook)

    out = pl.pallas_call(
        functools.partial(
            _p2_body, nt=nt, qg=qg, num_groups=num_groups
        ),
        grid=(num_groups,),
        in_specs=[
            pl.BlockSpec(memory_space=pltpu.SMEM),
            pl.BlockSpec((qg, d_dim), lambda i: (i, 0)),
            pl.BlockSpec((qg, 1), lambda i: (i, 0)),
            pl.BlockSpec(memory_space=pl.ANY),
        ],
        out_specs=pl.BlockSpec((qg, 1), lambda i: (i, 0)),
        out_shape=jax.ShapeDtypeStruct((q, 1), jnp.int32),
        scratch_shapes=[
            pltpu.VMEM((2, qg * nt, d_dim), jnp.float32),
            pltpu.SemaphoreType.DMA((2, qg)),
        ],
    )(bt, x, bt, codebook)
    return out
